# Initial kernel scaffold; baseline (speedup 1.0000x reference)
import jax
import jax.numpy as jnp
from jax.experimental import pallas as pl


def kernel(x, edge_index, edge_weight, W1, b1, W2, b2):
    raise NotImplementedError("write your pallas kernel here")


# speedup vs baseline: 5.8730x; 5.8730x over previous
import functools

import jax
import jax.numpy as jnp
from jax import lax
from jax.experimental import pallas as pl
from jax.experimental.pallas import tpu as pltpu
from jax.experimental.pallas import tpu_sc as plsc

N = 10000
E = 160000
IN_SIZE = 256
HID_SIZE = 512
OUT_SIZE = 128

NSC = 2
NT = 16
N_PAD = 10240
E_PAD = 163840
ROWS_T = N_PAD // NT
CHUNK = 128
EC = E_PAD // CHUNK
EC_T = EC // NT
EC_W = EC // (NSC * NT)
DEG_ROWS = N_PAD // 8

def _mesh():
    return plsc.VectorSubcoreMesh(
        core_axis_name="c", subcore_axis_name="s",
        num_cores=NSC, num_subcores=NT)


def _rsqrt16(d):
    d = d + 1.0
    yi = jnp.full((16,), 0x5F3759DF, jnp.int32) - lax.shift_right_logical(
        lax.bitcast_convert_type(d, jnp.int32), jnp.full((16,), 1, jnp.int32))
    y = lax.bitcast_convert_type(yi, jnp.float32)
    h = d * 0.5
    for _ in range(3):
        y = y * (1.5 - h * y * y)
    return y


def _norm_body(src_r, dst_r, ew_r,
               norm_out, dinv2_out,
               deg_sh, degbuf, deg_q, dstd_v, ewd_v, sidx_v, didx_v, ewn_v,
               dinv_v, dinv2_v, nrm_v, lan_v):
    c = lax.axis_index("c")
    s = lax.axis_index("s")
    t = s
    w = c * NT + s
    z16 = jnp.zeros((16,), jnp.float32)
    rows16 = lax.iota(jnp.int32, 16)
    lane16c = jnp.bitwise_and(rows16, jnp.full((16,), 7, jnp.int32)) * 16

    def z_step(r, carry):
        for k in range(8):
            degbuf[r, pl.ds(k * 16, 16)] = z16
        return carry
    lax.fori_loop(0, CHUNK, z_step, 0)
    pltpu.sync_copy(degbuf.at[pl.ds(0, DEG_ROWS // NT)],
                    deg_sh.at[pl.ds(t * (DEG_ROWS // NT), DEG_ROWS // NT)])
    pltpu.sync_copy(dst_r.at[pl.ds(t * EC_T, EC_T)], dstd_v)
    pltpu.sync_copy(ew_r.at[pl.ds(t * EC_T, EC_T)], ewd_v)
    def lz_step(g, carry):
        lan_v[pl.ds(g * 16, 16)] = jnp.zeros((16,), jnp.int32)
        return carry
    lax.fori_loop(0, CHUNK // 16, lz_step, 0)
    plsc.subcore_barrier()

    def deg_step(j, carry):
        for g in range(CHUNK // 16):
            gsl = pl.ds(g * 16, 16)
            di = dstd_v[j, gsl]
            lane = jnp.bitwise_and(di, jnp.full((16,), 7, jnp.int32)) * 16
            plsc.store_scatter(degbuf, [rows16 + g * 16, lan_v[gsl]], z16)
            plsc.store_scatter(degbuf, [rows16 + g * 16, lane],
                               ewd_v[j, gsl])
            lan_v[gsl] = lane
            dstd_v[j, gsl] = lax.shift_right_logical(
                di, jnp.full((16,), 3, jnp.int32))
        pltpu.sync_copy(degbuf, deg_sh.at[dstd_v.at[j]], add=True)
        return carry
    lax.fori_loop(0, EC_T, deg_step, 0)
    plsc.subcore_barrier()

    QROWS = 320
    for q in range(DEG_ROWS // QROWS):
        pltpu.sync_copy(deg_sh.at[pl.ds(q * QROWS, QROWS)], deg_q)

        def rs_step(i, carry):
            rloc = i * 2 + lax.shift_right_logical(
                rows16, jnp.full((16,), 3, jnp.int32))
            d = plsc.load_gather(deg_q, [rloc, lane16c])
            y = _rsqrt16(d)
            dinv_v[pl.ds(q * QROWS * 8 + i * 16, 16)] = y
            dinv2_v[pl.ds(q * QROWS * 8 + i * 16, 16)] = y * y
            return carry
        lax.fori_loop(0, QROWS * 8 // 16, rs_step, 0)

    @pl.when(c == 0)
    def _():
        pltpu.sync_copy(dinv2_v.at[pl.ds(t * ROWS_T, ROWS_T)],
                        dinv2_out.at[pl.ds(t * ROWS_T, ROWS_T)])

    pltpu.sync_copy(src_r.at[pl.ds(w * EC_W, EC_W)], sidx_v)
    pltpu.sync_copy(dst_r.at[pl.ds(w * EC_W, EC_W)], didx_v)
    pltpu.sync_copy(ew_r.at[pl.ds(w * EC_W, EC_W)], ewn_v)

    def n_step(r, carry):
        for k in range(CHUNK // 16):
            si = sidx_v[r, pl.ds(k * 16, 16)]
            di = didx_v[r, pl.ds(k * 16, 16)]
            ev = ewn_v[r, pl.ds(k * 16, 16)]
            a = plsc.load_gather(dinv_v, [si])
            b = plsc.load_gather(dinv_v, [di])
            nrm_v[r, pl.ds(k * 16, 16)] = a * ev * b
        return carry
    lax.fori_loop(0, EC_W, n_step, 0)
    pltpu.sync_copy(nrm_v, norm_out.at[pl.ds(w * EC_W, EC_W)])


@functools.cache
def _get_norm_call():
  return pl.kernel(
    _norm_body,
    out_type=(jax.ShapeDtypeStruct((EC, CHUNK), jnp.float32),
              jax.ShapeDtypeStruct((N_PAD,), jnp.float32)),
    mesh=_mesh(),
    compiler_params=pltpu.CompilerParams(needs_layout_passes=False),
    scratch_types=[
        pltpu.VMEM_SHARED((DEG_ROWS, 128), jnp.float32),
        pltpu.VMEM((CHUNK, 128), jnp.float32),
        pltpu.VMEM((320, 128), jnp.float32),
        pltpu.VMEM((EC_T, CHUNK), jnp.int32),
        pltpu.VMEM((EC_T, CHUNK), jnp.float32),
        pltpu.VMEM((EC_W, CHUNK), jnp.int32),
        pltpu.VMEM((EC_W, CHUNK), jnp.int32),
        pltpu.VMEM((EC_W, CHUNK), jnp.float32),
        pltpu.VMEM((N_PAD,), jnp.float32),
        pltpu.VMEM((N_PAD,), jnp.float32),
        pltpu.VMEM((EC_W, CHUNK), jnp.float32),
        pltpu.VMEM((CHUNK,), jnp.int32),
    ],
  )


def _agg_body(mat, src_r, dst_r, nrm_r,
              out, acc_sh, sidx_v, didx_v, nrm_v, buf, sem):
    n_chunks = EC_T
    c = lax.axis_index("c")
    s = lax.axis_index("s")
    t = s

    z16 = jnp.zeros((16,), jnp.float32)

    def z_step(r, carry):
        for k in range(8):
            buf[r, pl.ds(k * 16, 16)] = z16
        return carry
    lax.fori_loop(0, CHUNK, z_step, 0)
    for q in range(ROWS_T // CHUNK):
        pltpu.sync_copy(
            buf, acc_sh.at[pl.ds(t * ROWS_T + q * CHUNK, CHUNK)])
    pltpu.sync_copy(src_r.at[pl.ds(t * n_chunks, n_chunks)], sidx_v)
    pltpu.sync_copy(dst_r.at[pl.ds(t * n_chunks, n_chunks)], didx_v)
    pltpu.sync_copy(nrm_r.at[pl.ds(t * n_chunks, n_chunks)], nrm_v)

    @pl.when(c == 1)
    def _():
        def off_step(r, carry):
            for k in range(CHUNK // 16):
                sidx_v[r, pl.ds(k * 16, 16)] = (
                    sidx_v[r, pl.ds(k * 16, 16)]
                    + jnp.full((16,), N_PAD, jnp.int32))
            return carry
        lax.fori_loop(0, n_chunks, off_step, 0)
    plsc.subcore_barrier()

    def chunk_step(j, carry):
        pltpu.async_copy(mat.at[sidx_v.at[j]], buf, sem).wait()

        def edge_step(e, carry2):
            nv = plsc.load_gather(
                nrm_v, [jnp.full((16,), j, jnp.int32),
                        jnp.full((16,), e, jnp.int32)])
            for k in range(8):
                buf[e, pl.ds(k * 16, 16)] = buf[e, pl.ds(k * 16, 16)] * nv
            return carry2
        lax.fori_loop(0, CHUNK, edge_step, 0)
        pltpu.sync_copy(buf, acc_sh.at[didx_v.at[j]], add=True)
        return carry
    lax.fori_loop(0, EC_T, chunk_step, 0)
    plsc.subcore_barrier()
    pltpu.sync_copy(acc_sh.at[pl.ds(t * ROWS_T, ROWS_T)],
                    out.at[pl.ds(c * N_PAD + t * ROWS_T, ROWS_T)])


@functools.cache
def _make_agg():
    return pl.kernel(
        _agg_body,
        out_type=jax.ShapeDtypeStruct((NSC * N_PAD, 128), jnp.float32),
        mesh=_mesh(),
        compiler_params=pltpu.CompilerParams(needs_layout_passes=False),
        scratch_types=[
            pltpu.VMEM_SHARED((N_PAD, 128), jnp.float32),
            pltpu.VMEM((EC_T, CHUNK), jnp.int32),
            pltpu.VMEM((EC_T, CHUNK), jnp.int32),
            pltpu.VMEM((EC_T, CHUNK), jnp.float32),
            pltpu.VMEM((CHUNK, 128), jnp.float32),
            pltpu.SemaphoreType.DMA,
        ],
    )


MB = 512


def _mm_body(xa_agg, xb_agg, xa, xb, dinv2, w1a, w1b, b1, w2, xemb, hw2):
    ta = xa_agg[...] + dinv2[...] * xa[...]
    tb = xb_agg[...] + dinv2[...] * xb[...]
    xe = (jnp.dot(ta, w1a[...], preferred_element_type=jnp.float32)
          + jnp.dot(tb, w1b[...], preferred_element_type=jnp.float32)
          + b1[...])
    xemb[...] = xe
    hw2[...] = jnp.dot(jnp.maximum(xe, 0.0), w2[...],
                       preferred_element_type=jnp.float32)


def _fin_body(p0, hw2, dinv2, b2, out):
    out[...] = p0[...] + dinv2[...] * hw2[...] + b2[...]


def _row_spec(width):
    return pl.BlockSpec((MB, width), lambda i: (i, 0))


def _full_spec(r, cdim):
    return pl.BlockSpec((r, cdim), lambda i: (0, 0))


@functools.cache
def _get_mm_call():
    return pl.pallas_call(
        _mm_body,
        grid=(N_PAD // MB,),
        in_specs=[_row_spec(128), _row_spec(128), _row_spec(128),
                  _row_spec(128),
                  pl.BlockSpec((MB, 1), lambda i: (i, 0)),
                  _full_spec(128, HID_SIZE), _full_spec(128, HID_SIZE),
                  _full_spec(1, HID_SIZE), _full_spec(HID_SIZE, OUT_SIZE)],
        out_specs=[_row_spec(HID_SIZE), _row_spec(OUT_SIZE)],
        out_shape=[jax.ShapeDtypeStruct((N_PAD, HID_SIZE), jnp.float32),
                   jax.ShapeDtypeStruct((N_PAD, OUT_SIZE), jnp.float32)],
    )


@functools.cache
def _get_fin_call():
    return pl.pallas_call(
        _fin_body,
        grid=(N_PAD // MB,),
        in_specs=[_row_spec(OUT_SIZE), _row_spec(OUT_SIZE),
                  pl.BlockSpec((MB, 1), lambda i: (i, 0)),
                  _full_spec(1, OUT_SIZE)],
        out_specs=_row_spec(OUT_SIZE),
        out_shape=jax.ShapeDtypeStruct((N_PAD, OUT_SIZE), jnp.float32),
    )


def kernel(x, edge_index, edge_weight, W1, b1, W2, b2):
    ep = E_PAD - E
    src = jnp.concatenate([edge_index[0], jnp.zeros((ep,), jnp.int32)])
    dst = jnp.concatenate([edge_index[1], jnp.zeros((ep,), jnp.int32)])
    ew = jnp.concatenate([edge_weight, jnp.zeros((ep,), jnp.float32)])
    src_r = src.reshape(EC, CHUNK)
    dst_r = dst.reshape(EC, CHUNK)
    ew_r = ew.reshape(EC, CHUNK)
    norm, dinv2 = _get_norm_call()(src_r, dst_r, ew_r)

    xp = jnp.pad(x, ((0, N_PAD - N), (0, 0)))
    xab = xp.reshape(N_PAD, 2, 128).transpose(1, 0, 2).reshape(2 * N_PAD, 128)
    agg1 = _make_agg()(xab, src_r, dst_r, norm)

    dinv2_col = dinv2.reshape(N_PAD, 1)
    xemb_p, hw2_p = _get_mm_call()(
        agg1[:N_PAD], agg1[N_PAD:], xab[:N_PAD], xab[N_PAD:],
        dinv2_col, W1[:128], W1[128:], b1.reshape(1, HID_SIZE), W2)

    hw2_ab = jnp.concatenate([hw2_p, hw2_p], axis=0)
    agg2 = _make_agg()(hw2_ab, src_r, dst_r, norm)
    out_p = _get_fin_call()(agg2[:N_PAD], hw2_p, dinv2_col,
                            b2.reshape(1, OUT_SIZE))
    return (out_p[:N], xemb_p[:N])


# speedup vs baseline: 7.6033x; 1.2946x over previous
import functools

import jax
import jax.numpy as jnp
from jax import lax
from jax.experimental import pallas as pl
from jax.experimental.pallas import tpu as pltpu
from jax.experimental.pallas import tpu_sc as plsc

N = 10000
E = 160000
IN_SIZE = 256
HID_SIZE = 512
OUT_SIZE = 128

NSC = 2
NT = 16
N_PAD = 10240
E_PAD = 163840
ROWS_T = N_PAD // NT
CHUNK = 128
EC = E_PAD // CHUNK
EC_T = EC // NT
EC_W = EC // (NSC * NT)
DEG_ROWS = N_PAD // 8

def _mesh():
    return plsc.VectorSubcoreMesh(
        core_axis_name="c", subcore_axis_name="s",
        num_cores=NSC, num_subcores=NT)


def _rsqrt16(d):
    d = d + 1.0
    yi = jnp.full((16,), 0x5F3759DF, jnp.int32) - lax.shift_right_logical(
        lax.bitcast_convert_type(d, jnp.int32), jnp.full((16,), 1, jnp.int32))
    y = lax.bitcast_convert_type(yi, jnp.float32)
    h = d * 0.5
    for _ in range(3):
        y = y * (1.5 - h * y * y)
    return y


def _norm_body(src_r, dst_r, ew_r,
               norm_out, dinv2_out,
               deg_sh, degbuf, deg_q, dstd_v, ewd_v, sidx_v, didx_v, ewn_v,
               dinv_v, dinv2_v, nrm_v, lan_v):
    c = lax.axis_index("c")
    s = lax.axis_index("s")
    t = s
    w = c * NT + s
    z16 = jnp.zeros((16,), jnp.float32)
    rows16 = lax.iota(jnp.int32, 16)
    lane16c = jnp.bitwise_and(rows16, jnp.full((16,), 7, jnp.int32)) * 16

    def z_step(r, carry):
        for k in range(8):
            degbuf[r, pl.ds(k * 16, 16)] = z16
        return carry
    lax.fori_loop(0, CHUNK, z_step, 0)
    pltpu.sync_copy(degbuf.at[pl.ds(0, DEG_ROWS // NT)],
                    deg_sh.at[pl.ds(t * (DEG_ROWS // NT), DEG_ROWS // NT)])
    pltpu.sync_copy(dst_r.at[pl.ds(t * EC_T, EC_T)], dstd_v)
    pltpu.sync_copy(ew_r.at[pl.ds(t * EC_T, EC_T)], ewd_v)
    def lz_step(g, carry):
        lan_v[pl.ds(g * 16, 16)] = jnp.zeros((16,), jnp.int32)
        return carry
    lax.fori_loop(0, CHUNK // 16, lz_step, 0)
    plsc.subcore_barrier()

    def deg_step(j, carry):
        for g in range(CHUNK // 16):
            gsl = pl.ds(g * 16, 16)
            di = dstd_v[j, gsl]
            lane = jnp.bitwise_and(di, jnp.full((16,), 7, jnp.int32)) * 16
            plsc.store_scatter(degbuf, [rows16 + g * 16, lan_v[gsl]], z16)
            plsc.store_scatter(degbuf, [rows16 + g * 16, lane],
                               ewd_v[j, gsl])
            lan_v[gsl] = lane
            dstd_v[j, gsl] = lax.shift_right_logical(
                di, jnp.full((16,), 3, jnp.int32))
        pltpu.sync_copy(degbuf, deg_sh.at[dstd_v.at[j]], add=True)
        return carry
    lax.fori_loop(0, EC_T, deg_step, 0)
    plsc.subcore_barrier()

    QROWS = 320
    for q in range(DEG_ROWS // QROWS):
        pltpu.sync_copy(deg_sh.at[pl.ds(q * QROWS, QROWS)], deg_q)

        def rs_step(i, carry):
            rloc = i * 2 + lax.shift_right_logical(
                rows16, jnp.full((16,), 3, jnp.int32))
            d = plsc.load_gather(deg_q, [rloc, lane16c])
            y = _rsqrt16(d)
            dinv_v[pl.ds(q * QROWS * 8 + i * 16, 16)] = y
            dinv2_v[pl.ds(q * QROWS * 8 + i * 16, 16)] = y * y
            return carry
        lax.fori_loop(0, QROWS * 8 // 16, rs_step, 0)

    @pl.when(c == 0)
    def _():
        pltpu.sync_copy(dinv2_v.at[pl.ds(t * ROWS_T, ROWS_T)],
                        dinv2_out.at[pl.ds(t * ROWS_T, ROWS_T)])

    pltpu.sync_copy(src_r.at[pl.ds(w * EC_W, EC_W)], sidx_v)
    pltpu.sync_copy(dst_r.at[pl.ds(w * EC_W, EC_W)], didx_v)
    pltpu.sync_copy(ew_r.at[pl.ds(w * EC_W, EC_W)], ewn_v)

    def n_step(r, carry):
        for k in range(CHUNK // 16):
            si = sidx_v[r, pl.ds(k * 16, 16)]
            di = didx_v[r, pl.ds(k * 16, 16)]
            ev = ewn_v[r, pl.ds(k * 16, 16)]
            a = plsc.load_gather(dinv_v, [si])
            b = plsc.load_gather(dinv_v, [di])
            nrm_v[r, pl.ds(k * 16, 16)] = a * ev * b
        return carry
    lax.fori_loop(0, EC_W, n_step, 0)
    pltpu.sync_copy(nrm_v, norm_out.at[pl.ds(w * EC_W, EC_W)])


@functools.cache
def _get_norm_call():
  return pl.kernel(
    _norm_body,
    out_type=(jax.ShapeDtypeStruct((EC, CHUNK), jnp.float32),
              jax.ShapeDtypeStruct((N_PAD,), jnp.float32)),
    mesh=_mesh(),
    compiler_params=pltpu.CompilerParams(needs_layout_passes=False),
    scratch_types=[
        pltpu.VMEM_SHARED((DEG_ROWS, 128), jnp.float32),
        pltpu.VMEM((CHUNK, 128), jnp.float32),
        pltpu.VMEM((320, 128), jnp.float32),
        pltpu.VMEM((EC_T, CHUNK), jnp.int32),
        pltpu.VMEM((EC_T, CHUNK), jnp.float32),
        pltpu.VMEM((EC_W, CHUNK), jnp.int32),
        pltpu.VMEM((EC_W, CHUNK), jnp.int32),
        pltpu.VMEM((EC_W, CHUNK), jnp.float32),
        pltpu.VMEM((N_PAD,), jnp.float32),
        pltpu.VMEM((N_PAD,), jnp.float32),
        pltpu.VMEM((EC_W, CHUNK), jnp.float32),
        pltpu.VMEM((CHUNK,), jnp.int32),
    ],
  )


def _agg_body(mat, src_r, dst_r, nrm_r,
              out, acc_sh, sidx_v, didx_v, buf, buf1, nrm_c, nrm_c1,
              sem, sem1, semn, semn1):
    c = lax.axis_index("c")
    s = lax.axis_index("s")
    t = s

    z16 = jnp.zeros((16,), jnp.float32)

    def z_step(r, carry):
        for k in range(8):
            buf[r, pl.ds(k * 16, 16)] = z16
        return carry
    lax.fori_loop(0, CHUNK, z_step, 0)
    for q in range(ROWS_T // CHUNK):
        pltpu.sync_copy(
            buf, acc_sh.at[pl.ds(t * ROWS_T + q * CHUNK, CHUNK)])
    plsc.subcore_barrier()

    HALF = EC_T // 2

    def scale_scatter(jl, b, nc):
        def edge_step(e, carry2):
            for u in range(2):
                ee = e * 2 + u
                nv = plsc.load_gather(nc, [jnp.full((16,), ee, jnp.int32)])
                for k in range(8):
                    b[ee, pl.ds(k * 16, 16)] = (
                        b[ee, pl.ds(k * 16, 16)] * nv)
            return carry2
        lax.fori_loop(0, CHUNK // 2, edge_step, 0)
        pltpu.sync_copy(b, acc_sh.at[didx_v.at[jl]], add=True)

    for h in range(2):
        gbase = t * EC_T + h * HALF
        pltpu.sync_copy(src_r.at[pl.ds(gbase, HALF)], sidx_v)
        pltpu.sync_copy(dst_r.at[pl.ds(gbase, HALF)], didx_v)

        @pl.when(c == 1)
        def _():
            def off_step(r, carry):
                for k in range(CHUNK // 16):
                    sidx_v[r, pl.ds(k * 16, 16)] = (
                        sidx_v[r, pl.ds(k * 16, 16)]
                        + jnp.full((16,), N_PAD, jnp.int32))
                return carry
            lax.fori_loop(0, HALF, off_step, 0)

        pltpu.async_copy(mat.at[sidx_v.at[0]], buf, sem)
        pltpu.async_copy(nrm_r.at[gbase], nrm_c, semn)
        NPAIR = HALF // 2

        def pair_step(i, carry):
            j0 = i * 2
            j1 = j0 + 1
            pltpu.async_copy(mat.at[sidx_v.at[j1]], buf1, sem1)
            pltpu.async_copy(nrm_r.at[gbase + j1], nrm_c1, semn1)
            pltpu.make_async_copy(mat.at[sidx_v.at[j0]], buf, sem).wait()
            pltpu.make_async_copy(nrm_r.at[gbase], nrm_c, semn).wait()
            scale_scatter(j0, buf, nrm_c)

            @pl.when(i + 1 < NPAIR)
            def _():
                pltpu.async_copy(mat.at[sidx_v.at[j0 + 2]], buf, sem)
                pltpu.async_copy(nrm_r.at[gbase + j0 + 2], nrm_c, semn)
            pltpu.make_async_copy(mat.at[sidx_v.at[j1]], buf1, sem1).wait()
            pltpu.make_async_copy(nrm_r.at[gbase + j1], nrm_c1, semn1).wait()
            scale_scatter(j1, buf1, nrm_c1)
            return carry
        lax.fori_loop(0, NPAIR, pair_step, 0)
    plsc.subcore_barrier()
    pltpu.sync_copy(acc_sh.at[pl.ds(t * ROWS_T, ROWS_T)],
                    out.at[pl.ds(c * N_PAD + t * ROWS_T, ROWS_T)])


@functools.cache
def _make_agg():
    return pl.kernel(
        _agg_body,
        out_type=jax.ShapeDtypeStruct((NSC * N_PAD, 128), jnp.float32),
        mesh=_mesh(),
        compiler_params=pltpu.CompilerParams(needs_layout_passes=False),
        scratch_types=[
            pltpu.VMEM_SHARED((N_PAD, 128), jnp.float32),
            pltpu.VMEM((EC_T // 2, CHUNK), jnp.int32),
            pltpu.VMEM((EC_T // 2, CHUNK), jnp.int32),
            pltpu.VMEM((CHUNK, 128), jnp.float32),
            pltpu.VMEM((CHUNK, 128), jnp.float32),
            pltpu.VMEM((CHUNK,), jnp.float32),
            pltpu.VMEM((CHUNK,), jnp.float32),
            pltpu.SemaphoreType.DMA,
            pltpu.SemaphoreType.DMA,
            pltpu.SemaphoreType.DMA,
            pltpu.SemaphoreType.DMA,
        ],
    )


MB = 512


def _mm_body(xa_agg, xb_agg, xa, xb, dinv2, w1a, w1b, b1, w2, xemb, hw2):
    ta = xa_agg[...] + dinv2[...] * xa[...]
    tb = xb_agg[...] + dinv2[...] * xb[...]
    xe = (jnp.dot(ta, w1a[...], preferred_element_type=jnp.float32)
          + jnp.dot(tb, w1b[...], preferred_element_type=jnp.float32)
          + b1[...])
    xemb[...] = xe
    hw2[...] = jnp.dot(jnp.maximum(xe, 0.0), w2[...],
                       preferred_element_type=jnp.float32)


def _fin_body(p0, hw2, dinv2, b2, out):
    out[...] = p0[...] + dinv2[...] * hw2[...] + b2[...]


def _row_spec(width):
    return pl.BlockSpec((MB, width), lambda i: (i, 0))


def _full_spec(r, cdim):
    return pl.BlockSpec((r, cdim), lambda i: (0, 0))


@functools.cache
def _get_mm_call():
    return pl.pallas_call(
        _mm_body,
        grid=(N_PAD // MB,),
        in_specs=[_row_spec(128), _row_spec(128), _row_spec(128),
                  _row_spec(128),
                  pl.BlockSpec((MB, 1), lambda i: (i, 0)),
                  _full_spec(128, HID_SIZE), _full_spec(128, HID_SIZE),
                  _full_spec(1, HID_SIZE), _full_spec(HID_SIZE, OUT_SIZE)],
        out_specs=[_row_spec(HID_SIZE), _row_spec(OUT_SIZE)],
        out_shape=[jax.ShapeDtypeStruct((N_PAD, HID_SIZE), jnp.float32),
                   jax.ShapeDtypeStruct((N_PAD, OUT_SIZE), jnp.float32)],
    )


@functools.cache
def _get_fin_call():
    return pl.pallas_call(
        _fin_body,
        grid=(N_PAD // MB,),
        in_specs=[_row_spec(OUT_SIZE), _row_spec(OUT_SIZE),
                  pl.BlockSpec((MB, 1), lambda i: (i, 0)),
                  _full_spec(1, OUT_SIZE)],
        out_specs=_row_spec(OUT_SIZE),
        out_shape=jax.ShapeDtypeStruct((N_PAD, OUT_SIZE), jnp.float32),
    )


def kernel(x, edge_index, edge_weight, W1, b1, W2, b2):
    ep = E_PAD - E
    src = jnp.concatenate([edge_index[0], jnp.zeros((ep,), jnp.int32)])
    dst = jnp.concatenate([edge_index[1], jnp.zeros((ep,), jnp.int32)])
    ew = jnp.concatenate([edge_weight, jnp.zeros((ep,), jnp.float32)])
    src_r = src.reshape(EC, CHUNK)
    dst_r = dst.reshape(EC, CHUNK)
    ew_r = ew.reshape(EC, CHUNK)
    norm, dinv2 = _get_norm_call()(src_r, dst_r, ew_r)

    xp = jnp.pad(x, ((0, N_PAD - N), (0, 0)))
    xab = xp.reshape(N_PAD, 2, 128).transpose(1, 0, 2).reshape(2 * N_PAD, 128)
    agg1 = _make_agg()(xab, src_r, dst_r, norm)

    dinv2_col = dinv2.reshape(N_PAD, 1)
    xemb_p, hw2_p = _get_mm_call()(
        agg1[:N_PAD], agg1[N_PAD:], xab[:N_PAD], xab[N_PAD:],
        dinv2_col, W1[:128], W1[128:], b1.reshape(1, HID_SIZE), W2)

    hw2_ab = jnp.concatenate([hw2_p, hw2_p], axis=0)
    agg2 = _make_agg()(hw2_ab, src_r, dst_r, norm)
    out_p = _get_fin_call()(agg2[:N_PAD], hw2_p, dinv2_col,
                            b2.reshape(1, OUT_SIZE))
    return (out_p[:N], xemb_p[:N])
